# Initial kernel scaffold; baseline (speedup 1.0000x reference)
import jax
import jax.numpy as jnp
from jax.experimental import pallas as pl


def kernel(data, lengths):
    raise NotImplementedError("write your pallas kernel here")


# speedup vs baseline: 4.5961x; 4.5961x over previous
import jax
import jax.numpy as jnp
from jax.experimental import pallas as pl


_G = 32


def _shuffle_body(lens_ref, x_ref, out_ref):
    k = pl.program_id(0)
    g, b, two_d = out_ref.shape
    merged = jnp.concatenate([x_ref[:, 0], x_ref[:, 1]], axis=-1)
    t2 = k * g + jax.lax.broadcasted_iota(jnp.int32, (g, b, two_d), 0)
    lens3 = jnp.broadcast_to(lens_ref[...], (g, b, two_d))
    out_ref[...] = jnp.where(t2 < lens3, merged, 0.0)


def kernel(data, lengths):
    T, B, D = data.shape
    T2 = T - (T % 2)
    H = T2 // 2
    newlens = (lengths // 2).astype(jnp.int32)
    x = data[:T2].reshape(H, 2, B, D)
    lens3d = newlens.reshape(1, B, 1)

    out = pl.pallas_call(
        _shuffle_body,
        grid=(H // _G,),
        in_specs=[
            pl.BlockSpec((1, B, 1), lambda k: (0, 0, 0)),
            pl.BlockSpec((_G, 2, B, D), lambda k: (k, 0, 0, 0)),
        ],
        out_specs=pl.BlockSpec((_G, B, 2 * D), lambda k: (k, 0, 0)),
        out_shape=jax.ShapeDtypeStruct((H, B, 2 * D), data.dtype),
    )(lens3d, x)
    return out, newlens


# speedup vs baseline: 5.1870x; 1.1286x over previous
import jax
import jax.numpy as jnp
from jax.experimental import pallas as pl


_G = 128


def _shuffle_body(lens_ref, x_ref, out_ref):
    k = pl.program_id(0)
    g, b, two_d = out_ref.shape
    merged = jnp.concatenate([x_ref[:, 0], x_ref[:, 1]], axis=-1)
    t2 = k * g + jax.lax.broadcasted_iota(jnp.int32, (g, b, two_d), 0)
    lens3 = jnp.broadcast_to(lens_ref[...], (g, b, two_d))
    out_ref[...] = jnp.where(t2 < lens3, merged, 0.0)


def kernel(data, lengths):
    T, B, D = data.shape
    T2 = T - (T % 2)
    H = T2 // 2
    newlens = (lengths // 2).astype(jnp.int32)
    x = data[:T2].reshape(H, 2, B, D)
    lens3d = newlens.reshape(1, B, 1)

    out = pl.pallas_call(
        _shuffle_body,
        grid=(H // _G,),
        in_specs=[
            pl.BlockSpec((1, B, 1), lambda k: (0, 0, 0)),
            pl.BlockSpec((_G, 2, B, D), lambda k: (k, 0, 0, 0)),
        ],
        out_specs=pl.BlockSpec((_G, B, 2 * D), lambda k: (k, 0, 0)),
        out_shape=jax.ShapeDtypeStruct((H, B, 2 * D), data.dtype),
    )(lens3d, x)
    return out, newlens
